# baseline (speedup 1.0000x reference)
import jax
import jax.numpy as jnp
from jax.experimental import pallas as pl
from jax.experimental.pallas import tpu as pltpu

_N = 16384
_M = 1000


def _loss_body(s_ref, pred_ref, tgt_ref, out_ref):
    ridx = s_ref[0]
    labrow = (ridx // 128) % 8
    labcol = ridx % 128
    ti = jax.lax.broadcasted_iota(jnp.int32, (8, 128), 0)
    tj = jax.lax.broadcasted_iota(jnp.int32, (8, 128), 1)
    tmask = (ti == labrow) & (tj == labcol)
    label = jnp.sum(jnp.where(tmask, tgt_ref[...], 0))

    r = ridx % 8
    pi = jax.lax.broadcasted_iota(jnp.int32, (8, _M), 0)
    pj = jax.lax.broadcasted_iota(jnp.int32, (8, _M), 1)
    rows = pred_ref[...]
    rowmask = pi == r
    total = jnp.sum(jnp.where(rowmask, rows, 0.0))
    elem = jnp.sum(jnp.where(rowmask & (pj == label), rows, 0.0))
    out_ref[0, 0] = _M * elem - total


def kernel(pred, target, rand_idx):
    ridx = jnp.asarray(rand_idx, jnp.int32).reshape((1,))
    tgt2d = jnp.asarray(target, jnp.int32).reshape(128, 128)
    out = pl.pallas_call(
        _loss_body,
        grid_spec=pltpu.PrefetchScalarGridSpec(
            num_scalar_prefetch=1,
            grid=(1,),
            in_specs=[
                pl.BlockSpec((8, _M), lambda i, s: (s[0] // 8, 0)),
                pl.BlockSpec((8, 128), lambda i, s: (s[0] // 1024, (s[0] % 1024) // 128)),
            ],
            out_specs=pl.BlockSpec(memory_space=pltpu.SMEM),
        ),
        out_shape=jax.ShapeDtypeStruct((1, 1), jnp.float32),
    )(ridx, pred, tgt2d)
    return out.reshape(())


# speedup vs baseline: 22.2150x; 22.2150x over previous
import jax
import jax.numpy as jnp
from jax.experimental import pallas as pl
from jax.experimental.pallas import tpu as pltpu

_N = 16384
_M = 1000


def _loss_body(s_ref, predT_ref, tgt_ref, out_ref):
    ridx = s_ref[0]
    labrow = (ridx // 128) % 8
    labcol = ridx % 128
    ti = jax.lax.broadcasted_iota(jnp.int32, (8, 128), 0)
    tj = jax.lax.broadcasted_iota(jnp.int32, (8, 128), 1)
    label = jnp.sum(jnp.where((ti == labrow) & (tj == labcol), tgt_ref[...], 0))

    c = ridx % 128
    blk = predT_ref[...]
    pi = jax.lax.broadcasted_iota(jnp.int32, (_M, 128), 0)
    pj = jax.lax.broadcasted_iota(jnp.int32, (_M, 128), 1)
    colmask = pj == c
    total = jnp.sum(jnp.where(colmask, blk, 0.0))
    elem = jnp.sum(jnp.where(colmask & (pi == label), blk, 0.0))
    out_ref[0, 0] = _M * elem - total


def kernel(pred, target, rand_idx):
    ridx = jnp.asarray(rand_idx, jnp.int32).reshape((1,))
    predT = pred.T
    tgt2d = jnp.asarray(target, jnp.int32).reshape(128, 128)
    out = pl.pallas_call(
        _loss_body,
        grid_spec=pltpu.PrefetchScalarGridSpec(
            num_scalar_prefetch=1,
            grid=(1,),
            in_specs=[
                pl.BlockSpec((_M, 128), lambda i, s: (0, s[0] // 128)),
                pl.BlockSpec((8, 128), lambda i, s: (s[0] // 1024, (s[0] % 1024) // 128)),
            ],
            out_specs=pl.BlockSpec(memory_space=pltpu.SMEM),
        ),
        out_shape=jax.ShapeDtypeStruct((1, 1), jnp.float32),
    )(ridx, predT, tgt2d)
    return out.reshape(())
